# baseline (speedup 1.0000x reference)
import jax
import jax.numpy as jnp
from jax.experimental import pallas as pl
from jax.experimental.pallas import tpu as pltpu

_BM = 3280


def _add_row_kernel(idx_ref, table_ref, x_ref, o_ref):
    row = table_ref[idx_ref[0], :]
    o_ref[...] = x_ref[...] + row[None, :]


def kernel(embeds, embed_type, table):
    b, s, h = embeds.shape
    n = b * s
    x = embeds.reshape(n, h)
    idx = jnp.asarray(embed_type, dtype=jnp.int32).reshape(1)
    out = pl.pallas_call(
        _add_row_kernel,
        grid_spec=pltpu.PrefetchScalarGridSpec(
            num_scalar_prefetch=1,
            grid=(pl.cdiv(n, _BM),),
            in_specs=[
                pl.BlockSpec(table.shape, lambda i, idx_ref: (0, 0)),
                pl.BlockSpec((_BM, h), lambda i, idx_ref: (i, 0)),
            ],
            out_specs=pl.BlockSpec((_BM, h), lambda i, idx_ref: (i, 0)),
        ),
        out_shape=jax.ShapeDtypeStruct((n, h), embeds.dtype),
        compiler_params=pltpu.CompilerParams(
            dimension_semantics=("parallel",),
            vmem_limit_bytes=67108864,
        ),
    )(idx, table, x)
    return out.reshape(b, s, h)


# speedup vs baseline: 1.0173x; 1.0173x over previous
import jax
import jax.numpy as jnp
from jax.experimental import pallas as pl
from jax.experimental.pallas import tpu as pltpu

_BM = 3840


def _add_row_kernel(idx_ref, table_ref, x_ref, o_ref):
    row = table_ref[idx_ref[0], :]
    o_ref[...] = x_ref[...] + row[None, :]


def kernel(embeds, embed_type, table):
    b, s, h = embeds.shape
    n = b * s
    x = embeds.reshape(n, h)
    idx = jnp.asarray(embed_type, dtype=jnp.int32).reshape(1)
    out = pl.pallas_call(
        _add_row_kernel,
        grid_spec=pltpu.PrefetchScalarGridSpec(
            num_scalar_prefetch=1,
            grid=(pl.cdiv(n, _BM),),
            in_specs=[
                pl.BlockSpec(table.shape, lambda i, idx_ref: (0, 0)),
                pl.BlockSpec((_BM, h), lambda i, idx_ref: (i, 0)),
            ],
            out_specs=pl.BlockSpec((_BM, h), lambda i, idx_ref: (i, 0)),
        ),
        out_shape=jax.ShapeDtypeStruct((n, h), embeds.dtype),
        compiler_params=pltpu.CompilerParams(
            dimension_semantics=("parallel",),
            vmem_limit_bytes=67108864,
        ),
    )(idx, table, x)
    return out.reshape(b, s, h)
